# baseline (speedup 1.0000x reference)
import functools

import jax
import jax.numpy as jnp
from jax import lax
from jax.experimental import pallas as pl
from jax.experimental.pallas import tpu as pltpu
from jax.experimental.pallas import tpu_sc as plsc

N = 10000
E = 320000
CHUNK = 128
NCHUNK = 2560
EPAD = NCHUNK * CHUNK
NCORES = 2
NSUB = 16
NW = NCORES * NSUB
CPW = NCHUNK // NW
RPT = 624
NTAIL = N - NSUB * RPT
HALF = CPW // 2
CPL = 40
NLOADS0 = 3
NB = 2
LEAD = 1

_MESH = plsc.VectorSubcoreMesh(core_axis_name="c", subcore_axis_name="s")
_MESH1 = plsc.VectorSubcoreMesh(core_axis_name="c", subcore_axis_name="s",
                                num_cores=1)


def _counts_body(row_hbm, col_hbm, out_hbm,
                 rowbufs, colbufs, sidx, srca, srcb, zbuf, acc):
    cid = lax.axis_index("c")
    sid = lax.axis_index("s")
    w = cid * NSUB + sid

    lane = lax.iota(jnp.int32, 16)
    e0 = jnp.where(lane == 0, 1.0, 0.0)
    e1 = jnp.where(lane == 1, 1.0, 0.0)
    z16 = jnp.zeros((16,), jnp.float32)
    for r in range(16):
        for k in range(8):
            zbuf[r, pl.ds(k * 16, 16)] = z16

    def sbody(i, carry):
        srca[i, pl.ds(0, 16)] = e0
        srcb[i, pl.ds(0, 16)] = e1
        for k in range(1, 8):
            srca[i, pl.ds(k * 16, 16)] = z16
            srcb[i, pl.ds(k * 16, 16)] = z16
        return carry

    lax.fori_loop(0, CHUNK, sbody, 0)
    for t in range(RPT // 16):
        pltpu.sync_copy(zbuf, acc.at[pl.ds(sid * RPT + t * 16, 16)])

    @pl.when(sid == 0)
    def _():
        pltpu.sync_copy(zbuf, acc.at[pl.ds(NSUB * RPT, 16)])
        pltpu.sync_copy(zbuf, acc.at[pl.ds(NSUB * RPT + 16, 16)])

    plsc.subcore_barrier()

    def cbody(j, carry):
        for k in range(CHUNK // 16):
            rv = rowbufs.at[j][pl.ds(k * 16, 16)]
            cv = colbufs.at[j][pl.ds(k * 16, 16)]
            sidx[0, pl.ds(k * 16, 16)] = jnp.where(rv == cv, rv, N)
        pltpu.sync_copy(srca, acc.at[rowbufs.at[j]], add=True)
        pltpu.sync_copy(srcb, acc.at[sidx.at[0]], add=True)
        return carry

    for half in range(CPW // HALF):
        base = w * CPW + half * HALF
        pltpu.sync_copy(row_hbm.at[pl.ds(base, HALF)], rowbufs)
        pltpu.sync_copy(col_hbm.at[pl.ds(base, HALF)], colbufs)
        lax.fori_loop(0, HALF, cbody, 0)
    plsc.subcore_barrier()
    pltpu.sync_copy(acc.at[pl.ds(sid * RPT, RPT)],
                    out_hbm.at[cid, pl.ds(sid * RPT, RPT)])

    @pl.when(sid == 0)
    def _():
        pltpu.sync_copy(acc.at[pl.ds(NSUB * RPT, NTAIL)],
                        out_hbm.at[cid, pl.ds(NSUB * RPT, NTAIL)])


_counts = functools.partial(
    pl.kernel,
    out_type=jax.ShapeDtypeStruct((NCORES, N, 128), jnp.float32),
    mesh=_MESH,
    scratch_types=[
        pltpu.VMEM((HALF, CHUNK), jnp.int32),
        pltpu.VMEM((HALF, CHUNK), jnp.int32),
        pltpu.VMEM((1, CHUNK), jnp.int32),
        pltpu.VMEM((CHUNK, 128), jnp.float32),
        pltpu.VMEM((CHUNK, 128), jnp.float32),
        pltpu.VMEM((16, 128), jnp.float32),
        pltpu.VMEM_SHARED((N + 16, 128), jnp.float32),
    ],
)(_counts_body)


def _scatter_body(g_hbm, row_hbm, col_hbm, out_hbm,
                  rowbufs, colbufs, ring, zbuf, acc, gsems, ssems, feat):
    cid = lax.axis_index("c")
    sid = lax.axis_index("s")
    base0 = jnp.where(cid == 0, sid * (CPL * NLOADS0),
                      NSUB * CPL * NLOADS0 + sid * CPL)

    z16 = jnp.zeros((16,), jnp.float32)
    for r in range(zbuf.shape[0]):
        for k in range(feat // 16):
            zbuf[r, pl.ds(k * 16, 16)] = z16
    for t in range(RPT // zbuf.shape[0]):
        pltpu.sync_copy(zbuf, acc.at[pl.ds(sid * RPT + t * zbuf.shape[0],
                                           zbuf.shape[0])])

    @pl.when(sid == 0)
    def _():
        pltpu.sync_copy(zbuf, acc.at[pl.ds(NSUB * RPT, NTAIL)])

    plsc.subcore_barrier()

    for load in range(NLOADS0):

      @pl.when((cid == 0) | (load == 0))
      def _(load=load):
        base = base0 + load * CPL
        pltpu.sync_copy(row_hbm.at[pl.ds(base, CPL)], rowbufs)
        pltpu.sync_copy(col_hbm.at[pl.ds(base, CPL)], colbufs)
        for q in range(LEAD):
            pltpu.async_copy(g_hbm.at[rowbufs.at[q]], ring.at[q % NB],
                             gsems.at[q % NB])

        def mbody(i, carry):
            j = i * NB
            for b in range(NB):
                q = j + b
                pltpu.make_async_copy(g_hbm.at[rowbufs.at[q]], ring.at[b],
                                      gsems.at[b]).wait()
                pltpu.async_copy(ring.at[b], acc.at[colbufs.at[q]],
                                 ssems.at[b], add=True)
                qn = q + LEAD
                bn = (b + LEAD) % NB

                @pl.when(qn < CPL)
                def _():
                    @pl.when(q >= LEAD)
                    def _():
                        pltpu.make_async_copy(
                            ring.at[bn], acc.at[colbufs.at[q - LEAD]],
                            ssems.at[bn]).wait()

                    pltpu.async_copy(g_hbm.at[rowbufs.at[qn]], ring.at[bn],
                                     gsems.at[bn])
            return carry

        lax.fori_loop(0, CPL // NB, mbody, 0)
        for k in range(NB):
            q = CPL - NB + k
            pltpu.make_async_copy(ring.at[q % NB], acc.at[colbufs.at[q]],
                                  ssems.at[q % NB]).wait()

    plsc.subcore_barrier()
    pltpu.sync_copy(acc.at[pl.ds(sid * RPT, RPT)],
                    out_hbm.at[cid, pl.ds(sid * RPT, RPT)])

    @pl.when(sid == 0)
    def _():
        pltpu.sync_copy(acc.at[pl.ds(NSUB * RPT, NTAIL)],
                        out_hbm.at[cid, pl.ds(NSUB * RPT, NTAIL)])


def _make_scatter(feat):
    return functools.partial(
        pl.kernel,
        out_type=jax.ShapeDtypeStruct((NCORES, N, feat), jnp.float32),
        mesh=_MESH,
        scratch_types=[
            pltpu.VMEM((CPL, CHUNK), jnp.int32),
            pltpu.VMEM((CPL, CHUNK), jnp.int32),
            pltpu.VMEM((NB, CHUNK, feat), jnp.float32),
            pltpu.VMEM((16, feat), jnp.float32),
            pltpu.VMEM_SHARED((N, feat), jnp.float32),
            pltpu.SemaphoreType.DMA((NB,)),
            pltpu.SemaphoreType.DMA((NB,)),
        ],
    )(functools.partial(_scatter_body, feat=feat))


_scatter128 = _make_scatter(128)


def _tca_body(cnts_ref, x_ref, w1c_ref, w1b_ref, w1a_ref, b1_ref,
              nv_ref, ga_ref, xb_ref, xa_ref):
    rc = cnts_ref[0][:, 0:1] + cnts_ref[1][:, 0:1]
    sc = cnts_ref[0][:, 1:2] + cnts_ref[1][:, 1:2]
    deg = rc - sc
    dinv = jnp.where(deg > 0, lax.rsqrt(jnp.maximum(deg, 1e-12)), 0.0)
    d2 = dinv * dinv
    nv_ref[:, 0:1] = dinv
    nv_ref[:, 1:2] = d2
    nv_ref[:, 2:3] = dinv * sc
    nv_ref[:, 3:4] = d2 * sc
    x = x_ref[...]
    ga_ref[...] = dinv * jnp.dot(x, w1c_ref[...],
                                 preferred_element_type=jnp.float32)
    xb_ref[...] = dinv * jnp.dot(x, w1b_ref[...],
                                 preferred_element_type=jnp.float32)
    xa_ref[...] = jnp.dot(x, w1a_ref[...],
                          preferred_element_type=jnp.float32) + b1_ref[...]


def _tcc_body(nv_ref, xb_ref, ga_ref, sa_ref, gv_ref):
    d2 = nv_ref[:, 1:2]
    e2 = nv_ref[:, 3:4]
    gv_ref[...] = (xb_ref[...] - d2 * (sa_ref[0] + sa_ref[1])
                   + e2 * ga_ref[...])


def _tce_body(nv_ref, xa_ref, gv_ref, sv_ref, w2c_ref, h_ref, ga2_ref):
    dinv = nv_ref[:, 0:1]
    e1 = nv_ref[:, 2:3]
    h = jnp.maximum(xa_ref[...] - dinv * (sv_ref[0] + sv_ref[1])
                    + e1 * gv_ref[...], 0.0)
    h_ref[...] = h
    ga2_ref[...] = dinv * jnp.dot(h, w2c_ref[...],
                                  preferred_element_type=jnp.float32)


def _tcg_body(nv_ref, h_ref, ga2_ref, sa2_ref, w2b_ref, gv2_ref):
    dinv = nv_ref[:, 0:1]
    d2 = nv_ref[:, 1:2]
    e2 = nv_ref[:, 3:4]
    gv2_ref[...] = (dinv * jnp.dot(h_ref[...], w2b_ref[...],
                                   preferred_element_type=jnp.float32)
                    - d2 * (sa2_ref[0] + sa2_ref[1]) + e2 * ga2_ref[...])


def _tci_body(nv_ref, h_ref, gv2_ref, sv2_ref, w2a_ref, b2_ref, out_ref):
    dinv = nv_ref[:, 0:1]
    e1 = nv_ref[:, 2:3]
    out_ref[...] = (jnp.dot(h_ref[...], w2a_ref[...],
                            preferred_element_type=jnp.float32)
                    + b2_ref[...]
                    - dinv * (sv2_ref[0, :, :64] + sv2_ref[1, :, :64])
                    + e1 * gv2_ref[:, :64])


def _f32(*shapes):
    return [jax.ShapeDtypeStruct(s, jnp.float32) for s in shapes]


_tca = pl.pallas_call(
    _tca_body, out_shape=_f32((N, 4), (N, 128), (N, 128), (N, 128)))
_tcc = pl.pallas_call(_tcc_body, out_shape=_f32((N, 128))[0])
_tce = pl.pallas_call(_tce_body, out_shape=_f32((N, 128), (N, 128)))
_tcg = pl.pallas_call(_tcg_body, out_shape=_f32((N, 128))[0])
_tci = pl.pallas_call(_tci_body, out_shape=_f32((N, 64))[0])


def kernel(x, edge_index, W1, b1, W2, b2):
    row = edge_index[0].astype(jnp.int32)
    col = edge_index[1].astype(jnp.int32)
    pad = jnp.zeros((EPAD - E,), jnp.int32)
    row = jnp.concatenate([row, pad]).reshape(NCHUNK, CHUNK)
    col = jnp.concatenate([col, pad]).reshape(NCHUNK, CHUNK)

    W1c = 2.0 * W1[2]
    W1b = W1[1]
    W1a = W1[0] - W1[2]
    zpad = jnp.zeros((128, 64), jnp.float32)
    W2c = jnp.concatenate([2.0 * W2[2], zpad], axis=1)
    W2b = jnp.concatenate([W2[1], zpad], axis=1)
    W2a = W2[0] - W2[2]

    cnts = _counts(row, col)
    nv, ga, xb, xa = _tca(cnts, x, W1c, W1b, W1a, b1)
    sa = _scatter128(ga, row, col)
    gv = _tcc(nv, xb, ga, sa)
    sv = _scatter128(gv, row, col)
    h, ga2 = _tce(nv, xa, gv, sv, W2c)
    sa2 = _scatter128(ga2, row, col)
    gv2 = _tcg(nv, h, ga2, sa2, W2b)
    sv2 = _scatter128(gv2, row, col)
    return _tci(nv, h, gv2, sv2, W2a, b2)


# speedup vs baseline: 1.0082x; 1.0082x over previous
import functools

import jax
import jax.numpy as jnp
from jax import lax
from jax.experimental import pallas as pl
from jax.experimental.pallas import tpu as pltpu
from jax.experimental.pallas import tpu_sc as plsc

N = 10000
E = 320000
CHUNK = 128
NCHUNK = 2560
EPAD = NCHUNK * CHUNK
NCORES = 2
NSUB = 16
NW = NCORES * NSUB
CPW = NCHUNK // NW
RPT = 624
NTAIL = N - NSUB * RPT
HALF = CPW // 2
CPL = 40
NLOADS0 = 3
NB = 2
LEAD = 1

_MESH = plsc.VectorSubcoreMesh(core_axis_name="c", subcore_axis_name="s")
_MESH1 = plsc.VectorSubcoreMesh(core_axis_name="c", subcore_axis_name="s",
                                num_cores=1)


def _counts_body(row_hbm, col_hbm, out_hbm,
                 rowbufs, colbufs, sidx, srca, srcb, zbuf, acc, sema, semb):
    cid = lax.axis_index("c")
    sid = lax.axis_index("s")
    w = cid * NSUB + sid

    lane = lax.iota(jnp.int32, 16)
    e0 = jnp.where(lane == 0, 1.0, 0.0)
    e1 = jnp.where(lane == 1, 1.0, 0.0)
    z16 = jnp.zeros((16,), jnp.float32)
    for r in range(16):
        for k in range(8):
            zbuf[r, pl.ds(k * 16, 16)] = z16

    def sbody(i, carry):
        srca[i, pl.ds(0, 16)] = e0
        srcb[i, pl.ds(0, 16)] = e1
        for k in range(1, 8):
            srca[i, pl.ds(k * 16, 16)] = z16
            srcb[i, pl.ds(k * 16, 16)] = z16
        return carry

    lax.fori_loop(0, CHUNK, sbody, 0)
    for t in range(RPT // 16):
        pltpu.sync_copy(zbuf, acc.at[pl.ds(sid * RPT + t * 16, 16)])

    @pl.when(sid == 0)
    def _():
        pltpu.sync_copy(zbuf, acc.at[pl.ds(NSUB * RPT, 16)])
        pltpu.sync_copy(zbuf, acc.at[pl.ds(NSUB * RPT + 16, 16)])

    plsc.subcore_barrier()

    def cbody(j, carry):
        for k in range(CHUNK // 16):
            rv = rowbufs.at[j][pl.ds(k * 16, 16)]
            cv = colbufs.at[j][pl.ds(k * 16, 16)]
            sidx[0, pl.ds(k * 16, 16)] = jnp.where(rv == cv, rv, N)
        da = pltpu.async_copy(srca, acc.at[rowbufs.at[j]], sema, add=True)
        db = pltpu.async_copy(srcb, acc.at[sidx.at[0]], semb, add=True)
        da.wait()
        db.wait()
        return carry

    for half in range(CPW // HALF):
        base = w * CPW + half * HALF
        pltpu.sync_copy(row_hbm.at[pl.ds(base, HALF)], rowbufs)
        pltpu.sync_copy(col_hbm.at[pl.ds(base, HALF)], colbufs)
        lax.fori_loop(0, HALF, cbody, 0)
    plsc.subcore_barrier()
    pltpu.sync_copy(acc.at[pl.ds(sid * RPT, RPT)],
                    out_hbm.at[cid, pl.ds(sid * RPT, RPT)])

    @pl.when(sid == 0)
    def _():
        pltpu.sync_copy(acc.at[pl.ds(NSUB * RPT, NTAIL)],
                        out_hbm.at[cid, pl.ds(NSUB * RPT, NTAIL)])


_counts = functools.partial(
    pl.kernel,
    out_type=jax.ShapeDtypeStruct((NCORES, N, 128), jnp.float32),
    mesh=_MESH,
    scratch_types=[
        pltpu.VMEM((HALF, CHUNK), jnp.int32),
        pltpu.VMEM((HALF, CHUNK), jnp.int32),
        pltpu.VMEM((1, CHUNK), jnp.int32),
        pltpu.VMEM((CHUNK, 128), jnp.float32),
        pltpu.VMEM((CHUNK, 128), jnp.float32),
        pltpu.VMEM((16, 128), jnp.float32),
        pltpu.VMEM_SHARED((N + 16, 128), jnp.float32),
        pltpu.SemaphoreType.DMA,
        pltpu.SemaphoreType.DMA,
    ],
)(_counts_body)


def _scatter_body(g_hbm, row_hbm, col_hbm, out_hbm,
                  rowbufs, colbufs, ring, zbuf, acc, gsems, ssems, feat):
    cid = lax.axis_index("c")
    sid = lax.axis_index("s")
    base0 = jnp.where(cid == 0, sid * (CPL * NLOADS0),
                      NSUB * CPL * NLOADS0 + sid * CPL)

    z16 = jnp.zeros((16,), jnp.float32)
    for r in range(zbuf.shape[0]):
        for k in range(feat // 16):
            zbuf[r, pl.ds(k * 16, 16)] = z16
    for t in range(RPT // zbuf.shape[0]):
        pltpu.sync_copy(zbuf, acc.at[pl.ds(sid * RPT + t * zbuf.shape[0],
                                           zbuf.shape[0])])

    @pl.when(sid == 0)
    def _():
        pltpu.sync_copy(zbuf, acc.at[pl.ds(NSUB * RPT, NTAIL)])

    plsc.subcore_barrier()

    for load in range(NLOADS0):

      @pl.when((cid == 0) | (load == 0))
      def _(load=load):
        base = base0 + load * CPL
        pltpu.sync_copy(row_hbm.at[pl.ds(base, CPL)], rowbufs)
        pltpu.sync_copy(col_hbm.at[pl.ds(base, CPL)], colbufs)
        for q in range(LEAD):
            pltpu.async_copy(g_hbm.at[rowbufs.at[q]], ring.at[q % NB],
                             gsems.at[q % NB])

        def mbody(i, carry):
            j = i * NB
            for b in range(NB):
                q = j + b
                pltpu.make_async_copy(g_hbm.at[rowbufs.at[q]], ring.at[b],
                                      gsems.at[b]).wait()
                pltpu.async_copy(ring.at[b], acc.at[colbufs.at[q]],
                                 ssems.at[b], add=True)
                qn = q + LEAD
                bn = (b + LEAD) % NB

                @pl.when(qn < CPL)
                def _():
                    @pl.when(q >= LEAD)
                    def _():
                        pltpu.make_async_copy(
                            ring.at[bn], acc.at[colbufs.at[q - LEAD]],
                            ssems.at[bn]).wait()

                    pltpu.async_copy(g_hbm.at[rowbufs.at[qn]], ring.at[bn],
                                     gsems.at[bn])
            return carry

        lax.fori_loop(0, CPL // NB, mbody, 0)
        for k in range(NB):
            q = CPL - NB + k
            pltpu.make_async_copy(ring.at[q % NB], acc.at[colbufs.at[q]],
                                  ssems.at[q % NB]).wait()

    plsc.subcore_barrier()
    pltpu.sync_copy(acc.at[pl.ds(sid * RPT, RPT)],
                    out_hbm.at[cid, pl.ds(sid * RPT, RPT)])

    @pl.when(sid == 0)
    def _():
        pltpu.sync_copy(acc.at[pl.ds(NSUB * RPT, NTAIL)],
                        out_hbm.at[cid, pl.ds(NSUB * RPT, NTAIL)])


def _make_scatter(feat):
    return functools.partial(
        pl.kernel,
        out_type=jax.ShapeDtypeStruct((NCORES, N, feat), jnp.float32),
        mesh=_MESH,
        scratch_types=[
            pltpu.VMEM((CPL, CHUNK), jnp.int32),
            pltpu.VMEM((CPL, CHUNK), jnp.int32),
            pltpu.VMEM((NB, CHUNK, feat), jnp.float32),
            pltpu.VMEM((16, feat), jnp.float32),
            pltpu.VMEM_SHARED((N, feat), jnp.float32),
            pltpu.SemaphoreType.DMA((NB,)),
            pltpu.SemaphoreType.DMA((NB,)),
        ],
    )(functools.partial(_scatter_body, feat=feat))


_scatter128 = _make_scatter(128)


def _tca_body(cnts_ref, x_ref, w1c_ref, w1b_ref, w1a_ref, b1_ref,
              nv_ref, ga_ref, xb_ref, xa_ref):
    rc = cnts_ref[0][:, 0:1] + cnts_ref[1][:, 0:1]
    sc = cnts_ref[0][:, 1:2] + cnts_ref[1][:, 1:2]
    deg = rc - sc
    dinv = jnp.where(deg > 0, lax.rsqrt(jnp.maximum(deg, 1e-12)), 0.0)
    d2 = dinv * dinv
    nv_ref[:, 0:1] = dinv
    nv_ref[:, 1:2] = d2
    nv_ref[:, 2:3] = dinv * sc
    nv_ref[:, 3:4] = d2 * sc
    x = x_ref[...]
    ga_ref[...] = dinv * jnp.dot(x, w1c_ref[...],
                                 preferred_element_type=jnp.float32)
    xb_ref[...] = dinv * jnp.dot(x, w1b_ref[...],
                                 preferred_element_type=jnp.float32)
    xa_ref[...] = jnp.dot(x, w1a_ref[...],
                          preferred_element_type=jnp.float32) + b1_ref[...]


def _tcc_body(nv_ref, xb_ref, ga_ref, sa_ref, gv_ref):
    d2 = nv_ref[:, 1:2]
    e2 = nv_ref[:, 3:4]
    gv_ref[...] = (xb_ref[...] - d2 * (sa_ref[0] + sa_ref[1])
                   + e2 * ga_ref[...])


def _tce_body(nv_ref, xa_ref, gv_ref, sv_ref, w2c_ref, h_ref, ga2_ref):
    dinv = nv_ref[:, 0:1]
    e1 = nv_ref[:, 2:3]
    h = jnp.maximum(xa_ref[...] - dinv * (sv_ref[0] + sv_ref[1])
                    + e1 * gv_ref[...], 0.0)
    h_ref[...] = h
    ga2_ref[...] = dinv * jnp.dot(h, w2c_ref[...],
                                  preferred_element_type=jnp.float32)


def _tcg_body(nv_ref, h_ref, ga2_ref, sa2_ref, w2b_ref, gv2_ref):
    dinv = nv_ref[:, 0:1]
    d2 = nv_ref[:, 1:2]
    e2 = nv_ref[:, 3:4]
    gv2_ref[...] = (dinv * jnp.dot(h_ref[...], w2b_ref[...],
                                   preferred_element_type=jnp.float32)
                    - d2 * (sa2_ref[0] + sa2_ref[1]) + e2 * ga2_ref[...])


def _tci_body(nv_ref, h_ref, gv2_ref, sv2_ref, w2a_ref, b2_ref, out_ref):
    dinv = nv_ref[:, 0:1]
    e1 = nv_ref[:, 2:3]
    out_ref[...] = (jnp.dot(h_ref[...], w2a_ref[...],
                            preferred_element_type=jnp.float32)
                    + b2_ref[...]
                    - dinv * (sv2_ref[0, :, :64] + sv2_ref[1, :, :64])
                    + e1 * gv2_ref[:, :64])


def _f32(*shapes):
    return [jax.ShapeDtypeStruct(s, jnp.float32) for s in shapes]


_tca = pl.pallas_call(
    _tca_body, out_shape=_f32((N, 4), (N, 128), (N, 128), (N, 128)))
_tcc = pl.pallas_call(_tcc_body, out_shape=_f32((N, 128))[0])
_tce = pl.pallas_call(_tce_body, out_shape=_f32((N, 128), (N, 128)))
_tcg = pl.pallas_call(_tcg_body, out_shape=_f32((N, 128))[0])
_tci = pl.pallas_call(_tci_body, out_shape=_f32((N, 64))[0])


def kernel(x, edge_index, W1, b1, W2, b2):
    row = edge_index[0].astype(jnp.int32)
    col = edge_index[1].astype(jnp.int32)
    pad = jnp.zeros((EPAD - E,), jnp.int32)
    row = jnp.concatenate([row, pad]).reshape(NCHUNK, CHUNK)
    col = jnp.concatenate([col, pad]).reshape(NCHUNK, CHUNK)

    W1c = 2.0 * W1[2]
    W1b = W1[1]
    W1a = W1[0] - W1[2]
    zpad = jnp.zeros((128, 64), jnp.float32)
    W2c = jnp.concatenate([2.0 * W2[2], zpad], axis=1)
    W2b = jnp.concatenate([W2[1], zpad], axis=1)
    W2a = W2[0] - W2[2]

    cnts = _counts(row, col)
    nv, ga, xb, xa = _tca(cnts, x, W1c, W1b, W1a, b1)
    sa = _scatter128(ga, row, col)
    gv = _tcc(nv, xb, ga, sa)
    sv = _scatter128(gv, row, col)
    h, ga2 = _tce(nv, xa, gv, sv, W2c)
    sa2 = _scatter128(ga2, row, col)
    gv2 = _tcg(nv, h, ga2, sa2, W2b)
    sv2 = _scatter128(gv2, row, col)
    return _tci(nv, h, gv2, sv2, W2a, b2)
